# baseline (speedup 1.0000x reference)
import jax
import jax.numpy as jnp
from jax.experimental import pallas as pl
from jax.experimental.pallas import tpu as pltpu

N = 10000
D = 256
BM = 400
NB = N // BM


def _fused_kernel(a_ref, x0_full_ref, x0_row_ref, e1_ref, e2_ref,
                  osum_ref, e1_scratch):
    i = pl.program_id(0)

    @pl.when(i < NB)
    def _():
        e1_blk = jnp.dot(a_ref[...], x0_full_ref[...],
                         preferred_element_type=jnp.float32)
        e1_ref[...] = e1_blk
        e1_scratch[pl.ds(i * BM, BM), :] = e1_blk

    @pl.when(i >= NB)
    def _():
        j = i - NB
        e2_blk = jnp.dot(a_ref[...], e1_scratch[...],
                         preferred_element_type=jnp.float32)
        e2_ref[...] = e2_blk
        osum_ref[...] = (
            x0_row_ref[...] + e1_scratch[pl.ds(j * BM, BM), :] + e2_blk)


def kernel(encoder_adj, init_emb):
    a_spec = pl.BlockSpec((BM, N), lambda i: (i % NB, 0))
    x0_full_spec = pl.BlockSpec((N, D), lambda i: (0, 0))
    x0_row_spec = pl.BlockSpec(
        (BM, D), lambda i: (jnp.maximum(i - NB, 0), 0))
    e1_spec = pl.BlockSpec((BM, D), lambda i: (jnp.minimum(i, NB - 1), 0))
    out2_spec = pl.BlockSpec((BM, D), lambda i: (jnp.maximum(i - NB, 0), 0))

    e1, e2, summed = pl.pallas_call(
        _fused_kernel,
        grid=(2 * NB,),
        in_specs=[a_spec, x0_full_spec, x0_row_spec],
        out_specs=[e1_spec, out2_spec, out2_spec],
        out_shape=[
            jax.ShapeDtypeStruct((N, D), jnp.float32),
            jax.ShapeDtypeStruct((N, D), jnp.float32),
            jax.ShapeDtypeStruct((N, D), jnp.float32),
        ],
        scratch_shapes=[pltpu.VMEM((N, D), jnp.float32)],
    )(encoder_adj, init_emb, init_emb)

    return (summed, init_emb, e1, e2)


# speedup vs baseline: 1.0242x; 1.0242x over previous
import jax
import jax.numpy as jnp
from jax.experimental import pallas as pl
from jax.experimental.pallas import tpu as pltpu

N = 10000
D = 256
BM = 400
NB = N // BM


def _fused_kernel(a_ref, x0_full_ref, e1_ref, e2_ref,
                  osum_ref, e1_scratch):
    i = pl.program_id(0)

    @pl.when(i < NB)
    def _():
        e1_blk = jnp.dot(a_ref[...], x0_full_ref[...],
                         preferred_element_type=jnp.float32)
        e1_ref[...] = e1_blk
        e1_scratch[pl.ds(i * BM, BM), :] = e1_blk

    @pl.when(i >= NB)
    def _():
        j = i - NB
        e2_blk = jnp.dot(a_ref[...], e1_scratch[...],
                         preferred_element_type=jnp.float32)
        e2_ref[...] = e2_blk
        osum_ref[...] = (
            x0_full_ref[pl.ds(j * BM, BM), :]
            + e1_scratch[pl.ds(j * BM, BM), :] + e2_blk)


def kernel(encoder_adj, init_emb):
    a_spec = pl.BlockSpec((BM, N), lambda i: (i % NB, 0))
    x0_full_spec = pl.BlockSpec((N, D), lambda i: (0, 0))
    e1_spec = pl.BlockSpec((BM, D), lambda i: (jnp.minimum(i, NB - 1), 0))
    out2_spec = pl.BlockSpec((BM, D), lambda i: (jnp.maximum(i - NB, 0), 0))

    e1, e2, summed = pl.pallas_call(
        _fused_kernel,
        grid=(2 * NB,),
        in_specs=[a_spec, x0_full_spec],
        out_specs=[e1_spec, out2_spec, out2_spec],
        out_shape=[
            jax.ShapeDtypeStruct((N, D), jnp.float32),
            jax.ShapeDtypeStruct((N, D), jnp.float32),
            jax.ShapeDtypeStruct((N, D), jnp.float32),
        ],
        scratch_shapes=[pltpu.VMEM((N, D), jnp.float32)],
    )(encoder_adj, init_emb)

    return (summed, init_emb, e1, e2)
